# Initial kernel scaffold; baseline (speedup 1.0000x reference)
import jax
import jax.numpy as jnp
from jax.experimental import pallas as pl


def kernel(input, table):
    raise NotImplementedError("write your pallas kernel here")


# speedup vs baseline: 2.3244x; 2.3244x over previous
import math
import functools

import jax
import jax.numpy as jnp
from jax.experimental import pallas as pl
from jax.experimental.pallas import tpu as pltpu
from jax.experimental.pallas import tpu_sc as plsc

VOCAB = 100000
EMB = 64
MAX_LEN = 200
BATCH = 4096
SEQ = 200

WINDOW = 128
NUM_ROWS = SEQ * BATCH
NUM_WINDOWS = NUM_ROWS // WINDOW


def _positional_encoding():
    position = jnp.arange(0, MAX_LEN).astype(jnp.float32)[:, None]
    div_term = jnp.exp(
        jnp.arange(0, EMB, 2).astype(jnp.float32) * -(math.log(10000.0) / EMB)
    )
    pe = jnp.zeros((MAX_LEN, EMB), dtype=jnp.float32)
    pe = pe.at[:, 0::2].set(jnp.sin(position * div_term))
    pe = pe.at[:, 1::2].set(jnp.cos(position * div_term))
    return pe


def _make_sc_kernel():
    mesh = plsc.VectorSubcoreMesh(core_axis_name="core", subcore_axis_name="subcore")

    @functools.partial(
        pl.kernel,
        out_type=jax.ShapeDtypeStruct((NUM_ROWS, EMB), jnp.float32),
        mesh=mesh,
        compiler_params=pltpu.CompilerParams(use_tc_tiling_on_sc=False),
    )
    def sc_kernel(table_hbm, idx_hbm, pe_hbm, out_hbm):
        def body(i_vmem, pe_vmem, o_vmem):
            pltpu.sync_copy(table_hbm.at[i_vmem.at[0]], o_vmem)
            pe_regs = [pe_vmem[0, pl.ds(16 * j, 16)] for j in range(EMB // 16)]

            @pl.loop(0, WINDOW)
            def _(r):
                for j in range(EMB // 16):
                    slc = pl.ds(16 * j, 16)
                    o_vmem[r, slc] = o_vmem[r, slc] + pe_regs[j]

        pltpu.emit_pipeline(
            body,
            grid=(NUM_WINDOWS,),
            in_specs=[
                pl.BlockSpec((1, WINDOW), index_map=lambda i: (0, i)),
                pl.BlockSpec((1, EMB), index_map=lambda i: (i, 0)),
            ],
            out_specs=[
                pl.BlockSpec((WINDOW, EMB), index_map=lambda i: (i, 0)),
            ],
            core_axis_name=("core", "subcore"),
            dimension_semantics=(pltpu.PARALLEL,),
        )(idx_hbm, pe_hbm, out_hbm)

    return sc_kernel


_SC_KERNEL = _make_sc_kernel()


def kernel(input, table):
    idx_t = input.T.reshape(1, NUM_ROWS).astype(jnp.int32)
    pe = _positional_encoding()
    pe_win = jnp.repeat(pe, BATCH // WINDOW, axis=0)
    out_flat = _SC_KERNEL(table, idx_t, pe_win)
    return out_flat.reshape(SEQ, BATCH, EMB)


# speedup vs baseline: 4.1651x; 1.7919x over previous
import math
import functools

import jax
import jax.numpy as jnp
from jax.experimental import pallas as pl
from jax.experimental.pallas import tpu as pltpu
from jax.experimental.pallas import tpu_sc as plsc

VOCAB = 100000
EMB = 64
MAX_LEN = 200
BATCH = 4096
SEQ = 200

GATHER = 128
WINDOW = 512
NUM_ROWS = SEQ * BATCH
NUM_WINDOWS = NUM_ROWS // WINDOW


def _positional_encoding():
    position = jnp.arange(0, MAX_LEN).astype(jnp.float32)[:, None]
    div_term = jnp.exp(
        jnp.arange(0, EMB, 2).astype(jnp.float32) * -(math.log(10000.0) / EMB)
    )
    pe = jnp.zeros((MAX_LEN, EMB), dtype=jnp.float32)
    pe = pe.at[:, 0::2].set(jnp.sin(position * div_term))
    pe = pe.at[:, 1::2].set(jnp.cos(position * div_term))
    return pe


def _make_sc_kernel():
    mesh = plsc.VectorSubcoreMesh(core_axis_name="core", subcore_axis_name="subcore")

    @functools.partial(
        pl.kernel,
        out_type=jax.ShapeDtypeStruct((NUM_ROWS, EMB), jnp.float32),
        mesh=mesh,
        compiler_params=pltpu.CompilerParams(use_tc_tiling_on_sc=False),
        scratch_types=[pltpu.SemaphoreType.DMA],
    )
    def sc_kernel(table_hbm, idx_hbm, pe_hbm, out_hbm, gsem):
        def body(i_vmem, pe_vmem, o_vmem):
            for j in range(WINDOW // GATHER):
                pltpu.async_copy(
                    table_hbm.at[i_vmem.at[0, pl.ds(j * GATHER, GATHER)]],
                    o_vmem.at[pl.ds(j * GATHER, GATHER), :],
                    gsem,
                )
            pltpu.make_async_copy(
                table_hbm.at[i_vmem.at[0]], o_vmem, gsem
            ).wait()
            pe_regs = [pe_vmem[0, pl.ds(16 * j, 16)] for j in range(EMB // 16)]

            @pl.loop(0, WINDOW, unroll=8)
            def _(r):
                for j in range(EMB // 16):
                    slc = pl.ds(16 * j, 16)
                    o_vmem[r, slc] = o_vmem[r, slc] + pe_regs[j]

        pltpu.emit_pipeline(
            body,
            grid=(NUM_WINDOWS,),
            in_specs=[
                pl.BlockSpec((1, WINDOW), index_map=lambda i: (0, i)),
                pl.BlockSpec((1, EMB), index_map=lambda i: (i, 0)),
            ],
            out_specs=[
                pl.BlockSpec((WINDOW, EMB), index_map=lambda i: (i, 0)),
            ],
            core_axis_name=("core", "subcore"),
            dimension_semantics=(pltpu.PARALLEL,),
        )(idx_hbm, pe_hbm, out_hbm)

    return sc_kernel


_SC_KERNEL = _make_sc_kernel()


def kernel(input, table):
    idx_t = input.T.reshape(1, NUM_ROWS).astype(jnp.int32)
    pe = _positional_encoding()
    pe_win = jnp.repeat(pe, BATCH // WINDOW, axis=0)
    out_flat = _SC_KERNEL(table, idx_t, pe_win)
    return out_flat.reshape(SEQ, BATCH, EMB)
